# baseline (speedup 1.0000x reference)
import jax
import jax.numpy as jnp
from jax import lax
from jax.experimental import pallas as pl
from jax.experimental.pallas import tpu as pltpu

ROWS = 128
COLS = 32768
RB = 8
NBLK = ROWS // RB
LANES = 128
NCH = COLS // LANES
NACC = 8
NBUF = 6
I32_MAX = 2147483647


HALF = COLS // 2


def _blk_copies(x_any, big, sems, i, slot):
    return [
        pltpu.make_async_copy(
            x_any.at[pl.ds(i * RB, RB), pl.ds(h * HALF, HALF)],
            big.at[pl.ds(slot * RB, RB), pl.ds(h * HALF, HALF)],
            sems.at[slot, h])
        for h in range(2)
    ]


def _argmax_body(x_any, o_ref, big, sems):
    for i in range(NBUF):
        for cp in _blk_copies(x_any, big, sems, i, i):
            cp.start()

    def step(i, _):
        slot = lax.rem(i, NBUF)
        for cp in _blk_copies(x_any, big, sems, i, slot):
            cp.wait()
        base = slot * RB

        bests = []
        bidxs = []
        for a in range(NACC):
            bests.append(big[pl.ds(base, RB), a * LANES:(a + 1) * LANES])
            bidxs.append(jnp.full((RB, LANES), a, jnp.int32))
        for k in range(NACC, NCH):
            a = k % NACC
            v = big[pl.ds(base, RB), k * LANES:(k + 1) * LANES]
            m = v > bests[a]
            bests[a] = jnp.where(m, v, bests[a])
            bidxs[a] = jnp.where(m, k, bidxs[a])

        n = NACC
        while n > 1:
            for a in range(n // 2):
                vl, il = bests[2 * a], bidxs[2 * a]
                vr, ir = bests[2 * a + 1], bidxs[2 * a + 1]
                m = (vr > vl) | ((vr == vl) & (ir < il))
                bests[a] = jnp.where(m, vr, vl)
                bidxs[a] = jnp.where(m, ir, il)
            n //= 2
        best, bidx = bests[0], bidxs[0]

        lanei = lax.broadcasted_iota(jnp.int32, (RB, LANES), 1)
        elem = (bidx << 7) | lanei
        mx = jnp.max(best, axis=1, keepdims=True)
        cand = jnp.where(best == mx, elem, I32_MAX)
        o_ref[pl.ds(i * RB, RB), :] = jnp.min(cand, axis=1, keepdims=True)

        @pl.when(i + NBUF < NBLK)
        def _():
            for cp in _blk_copies(x_any, big, sems, i + NBUF, slot):
                cp.start()

        return 0

    lax.fori_loop(0, NBLK, step, 0)


@jax.jit
def _argmax_tc(x):
    return pl.pallas_call(
        _argmax_body,
        in_specs=[pl.BlockSpec(memory_space=pl.ANY)],
        out_specs=pl.BlockSpec(memory_space=pltpu.VMEM),
        out_shape=jax.ShapeDtypeStruct((ROWS, 1), jnp.int32),
        scratch_shapes=[
            pltpu.VMEM((NBUF * RB, COLS), jnp.float32),
            pltpu.SemaphoreType.DMA((NBUF, 2)),
        ],
    )(x)


def kernel(inputs):
    return _argmax_tc(inputs)


# speedup vs baseline: 1.0060x; 1.0060x over previous
import jax
import jax.numpy as jnp
from jax import lax
from jax.experimental import pallas as pl
from jax.experimental.pallas import tpu as pltpu

ROWS = 128
COLS = 32768
RB = 8
NBLK = ROWS // RB
LANES = 128
NCH = COLS // LANES
NACC = 8
NBUF = 6
I32_MAX = 2147483647


def _blk_copy(x_any, big, sems, i, slot):
    return pltpu.make_async_copy(
        x_any.at[pl.ds(i * RB, RB)],
        big.at[pl.ds(slot * RB, RB)],
        sems.at[slot])


def _argmax_body(x_any, o_ref, big, sems):
    for i in range(NBUF):
        _blk_copy(x_any, big, sems, i, i).start()

    def step(i, _):
        slot = lax.rem(i, NBUF)
        _blk_copy(x_any, big, sems, i, slot).wait()
        base = slot * RB

        bests = []
        bidxs = []
        for a in range(NACC):
            bests.append(big[pl.ds(base, RB), a * LANES:(a + 1) * LANES])
            bidxs.append(jnp.full((RB, LANES), a, jnp.int32))
        for k in range(NACC, NCH):
            a = k % NACC
            v = big[pl.ds(base, RB), k * LANES:(k + 1) * LANES]
            m = v > bests[a]
            bests[a] = jnp.where(m, v, bests[a])
            bidxs[a] = jnp.where(m, k, bidxs[a])

        n = NACC
        while n > 1:
            for a in range(n // 2):
                vl, il = bests[2 * a], bidxs[2 * a]
                vr, ir = bests[2 * a + 1], bidxs[2 * a + 1]
                m = (vr > vl) | ((vr == vl) & (ir < il))
                bests[a] = jnp.where(m, vr, vl)
                bidxs[a] = jnp.where(m, ir, il)
            n //= 2
        best, bidx = bests[0], bidxs[0]

        lanei = lax.broadcasted_iota(jnp.int32, (RB, LANES), 1)
        elem = (bidx << 7) | lanei
        mx = jnp.max(best, axis=1, keepdims=True)
        cand = jnp.where(best == mx, elem, I32_MAX)
        o_ref[pl.ds(i * RB, RB), :] = jnp.min(cand, axis=1, keepdims=True)

        @pl.when(i + NBUF < NBLK)
        def _():
            _blk_copy(x_any, big, sems, i + NBUF, slot).start()

        return 0

    lax.fori_loop(0, NBLK, step, 0)


@jax.jit
def _argmax_tc(x):
    return pl.pallas_call(
        _argmax_body,
        in_specs=[pl.BlockSpec(memory_space=pl.ANY)],
        out_specs=pl.BlockSpec(memory_space=pltpu.VMEM),
        out_shape=jax.ShapeDtypeStruct((ROWS, 1), jnp.int32),
        scratch_shapes=[
            pltpu.VMEM((NBUF * RB, COLS), jnp.float32),
            pltpu.SemaphoreType.DMA((NBUF,)),
        ],
    )(x)


def kernel(inputs):
    return _argmax_tc(inputs)


# speedup vs baseline: 1.1121x; 1.1054x over previous
import jax
import jax.numpy as jnp
from jax import lax
from jax.experimental import pallas as pl
from jax.experimental.pallas import tpu as pltpu

ROWS = 128
COLS = 32768
RB = 8
NBLK = ROWS // RB
LANES = 128
NCH = COLS // LANES
NACC = 8
NBUF = 6
I32_MAX = 2147483647


def _blk_copy(x_any, big, sems, i, slot):
    return pltpu.make_async_copy(
        x_any.at[pl.ds(i * RB, RB)],
        big.at[pl.ds(slot * RB, RB)],
        sems.at[slot])


def _argmax_body(x_any, o_ref, big, sems, bestacc, elemacc):
    for i in range(NBUF):
        _blk_copy(x_any, big, sems, i, i).start()

    def step(i, _):
        slot = lax.rem(i, NBUF)
        _blk_copy(x_any, big, sems, i, slot).wait()
        base = slot * RB

        bests = []
        bidxs = []
        for a in range(NACC):
            bests.append(big[pl.ds(base, RB), a * LANES:(a + 1) * LANES])
            bidxs.append(jnp.full((RB, LANES), a, jnp.int32))
        for k in range(NACC, NCH):
            a = k % NACC
            v = big[pl.ds(base, RB), k * LANES:(k + 1) * LANES]
            m = v > bests[a]
            bests[a] = jnp.where(m, v, bests[a])
            bidxs[a] = jnp.where(m, k, bidxs[a])

        n = NACC
        while n > 1:
            for a in range(n // 2):
                vl, il = bests[2 * a], bidxs[2 * a]
                vr, ir = bests[2 * a + 1], bidxs[2 * a + 1]
                m = (vr > vl) | ((vr == vl) & (ir < il))
                bests[a] = jnp.where(m, vr, vl)
                bidxs[a] = jnp.where(m, ir, il)
            n //= 2
        best, bidx = bests[0], bidxs[0]

        lanei = lax.broadcasted_iota(jnp.int32, (RB, LANES), 1)
        elem = (bidx << 7) | lanei
        bestacc[pl.ds(i * RB, RB), :] = best
        elemacc[pl.ds(i * RB, RB), :] = elem

        @pl.when(i + NBUF < NBLK)
        def _():
            _blk_copy(x_any, big, sems, i + NBUF, slot).start()

        return 0

    lax.fori_loop(0, NBLK, step, 0)

    best = bestacc[...]
    elem = elemacc[...]
    mx = jnp.max(best, axis=1, keepdims=True)
    cand = jnp.where(best == mx, elem, I32_MAX)
    o_ref[...] = jnp.min(cand, axis=1, keepdims=True)


@jax.jit
def _argmax_tc(x):
    return pl.pallas_call(
        _argmax_body,
        in_specs=[pl.BlockSpec(memory_space=pl.ANY)],
        out_specs=pl.BlockSpec(memory_space=pltpu.VMEM),
        out_shape=jax.ShapeDtypeStruct((ROWS, 1), jnp.int32),
        scratch_shapes=[
            pltpu.VMEM((NBUF * RB, COLS), jnp.float32),
            pltpu.SemaphoreType.DMA((NBUF,)),
            pltpu.VMEM((ROWS, LANES), jnp.float32),
            pltpu.VMEM((ROWS, LANES), jnp.int32),
        ],
    )(x)


def kernel(inputs):
    return _argmax_tc(inputs)
